# baseline (speedup 1.0000x reference)
import jax
import jax.numpy as jnp
from jax import lax
from jax.experimental import pallas as pl
from jax.experimental.pallas import tpu as pltpu
from jax.experimental.pallas import tpu_sc as plsc

N = 10000
D = 128
E = 320000
E2 = 2 * E
NC = 2
NS = 16
NW = NC * NS
EPW = E2 // NW
RPW = 313
NP = NW * RPW
CH = 2000
NCHUNK = EPW // CH
DB = 80
GPC = CH // DB
VPG = DB // 16
RB = 128
RPAD = E2 + 2 * RB
NEG = -3.0e38

_mesh = plsc.VectorSubcoreMesh(core_axis_name="c", subcore_axis_name="s")
_cp = pltpu.CompilerParams(needs_layout_passes=False)


def _wid():
  return lax.axis_index("s") * NC + lax.axis_index("c")


def _iota16():
  return lax.iota(jnp.int32, 16)


def _extract(vec, pos):
  return jnp.sum(jnp.where(_iota16() == pos, vec, 0))


def _zero_i32(ref, nwords):
  def body(i, _):
    ref[pl.ds(i * 16, 16)] = jnp.zeros((16,), jnp.int32)
    return 0
  lax.fori_loop(0, nwords // 16, body, 0)


def _fill_f32(ref, nwords, val):
  def body(i, _):
    ref[pl.ds(i * 16, 16)] = jnp.full((16,), val, jnp.float32)
    return 0
  lax.fori_loop(0, nwords // 16, body, 0)


def _fill_i32(ref, nwords, val):
  def body(i, _):
    ref[pl.ds(i * 16, 16)] = jnp.full((16,), val, jnp.int32)
    return 0
  lax.fori_loop(0, nwords // 16, body, 0)


def _p1_body(src_hbm, dst_hbm, hist_hbm, sbuf, dbuf, hbuf):
  w = _wid()
  swap = w >= 16
  jbase = (w % 16) * EPW

  _zero_i32(hbuf, NP)

  def chunk(k, _):
    off = jbase + k * CH
    pltpu.sync_copy(src_hbm.at[pl.ds(off, CH)], sbuf)
    pltpu.sync_copy(dst_hbm.at[pl.ds(off, CH)], dbuf)

    def vreg(i, _):
      s = sbuf[pl.ds(i * 16, 16)]
      d = dbuf[pl.ds(i * 16, 16)]
      row = jnp.where(swap, d, s)
      cnt, last = plsc.scan_count(row)
      old = plsc.load_gather(hbuf, [row], mask=last)
      plsc.store_scatter(hbuf, [row], old + cnt, mask=last)
      return 0
    lax.fori_loop(0, CH // 16, vreg, 0)
    return 0
  lax.fori_loop(0, NCHUNK, chunk, 0)

  pltpu.sync_copy(hbuf, hist_hbm.at[w])


def _p2_body(src_hbm, dst_hbm, hist_hbm, r_hbm, rs_hbm,
             sbuf, dbuf, recb0, recb1, posb0, posb1, tbuf, cbuf, pbuf,
             tmpb0, tmpb1, rs2, sem0, sem1, hsem0, hsem1):
  w = _wid()
  swap = w >= 16
  jbase = (w % 16) * EPW

  _zero_i32(tbuf, NP)
  _zero_i32(cbuf, NP)

  tmps = (tmpb0, tmpb1)
  hsems = (hsem0, hsem1)
  pltpu.async_copy(hist_hbm.at[0], tmps[0], hsems[0])
  for wp in range(NW):
    if wp + 1 < NW:
      pltpu.async_copy(hist_hbm.at[wp + 1], tmps[(wp + 1) % 2],
                       hsems[(wp + 1) % 2])
    tmpb = tmps[wp % 2]
    pltpu.make_async_copy(hist_hbm.at[wp], tmpb, hsems[wp % 2]).wait()

    def addrow(i, _):
      t = tmpb[pl.ds(i * 16, 16)]
      tbuf[pl.ds(i * 16, 16)] = tbuf[pl.ds(i * 16, 16)] + t
      cbuf[pl.ds(i * 16, 16)] = cbuf[pl.ds(i * 16, 16)] + jnp.where(
          wp < w, t, 0)
      return 0
    lax.fori_loop(0, NP // 16, addrow, 0, unroll=4)

  def prefix(i, carry):
    t = tbuf[pl.ds(i * 16, 16)]
    cs = plsc.cumsum(t)
    excl = cs - t + carry
    pbuf[pl.ds(i * 16, 16)] = excl
    cbuf[pl.ds(i * 16, 16)] = cbuf[pl.ds(i * 16, 16)] + excl
    return carry + jnp.max(cs)
  lax.fori_loop(0, NP // 16, prefix, 0)

  @pl.when(w == 0)
  def _():
    io = _iota16()
    for half in range(2):
      starts = plsc.load_gather(pbuf, [(io + half * 16) * RPW])
      nxt = jnp.minimum((io + half * 16 + 1) * RPW, NP - 1)
      ends = plsc.load_gather(pbuf, [nxt])
      if half == 1:
        ends = jnp.where(io == 15, E2, ends)
      rs2[0, pl.ds(half * 16, 16)] = starts
      rs2[1, pl.ds(half * 16, 16)] = ends
    pltpu.sync_copy(rs2, rs_hbm)

  def chunk_impl(k, recb, posb, sem):
    off = jbase + k * CH
    pltpu.sync_copy(src_hbm.at[pl.ds(off, CH)], sbuf)
    pltpu.sync_copy(dst_hbm.at[pl.ds(off, CH)], dbuf)

    @pl.when(k >= 2)
    def _():
      for g in range(GPC):
        pltpu.make_async_copy(recb.at[g], r_hbm.at[posb.at[g]], sem).wait()

    for g in range(GPC):
      def vreg(v, _):
        i = g * VPG + v
        s = sbuf[pl.ds(i * 16, 16)]
        d = dbuf[pl.ds(i * 16, 16)]
        row = jnp.where(swap, d, s)
        nb = jnp.where(swap, s, d)
        recb[g, pl.ds(v * 16, 16)] = row * 32768 + nb * 2
        cnt, last = plsc.scan_count(row)
        base = plsc.load_gather(cbuf, [row])
        posb[g, pl.ds(v * 16, 16)] = base + cnt - 1
        plsc.store_scatter(cbuf, [row], base + cnt, mask=last)
        return 0
      lax.fori_loop(0, VPG, vreg, 0)

    for g in range(GPC):
      pltpu.async_copy(recb.at[g], r_hbm.at[posb.at[g]], sem)

  def chunk(k, _):
    @pl.when(k % 2 == 0)
    def _():
      chunk_impl(k, recb0, posb0, sem0)

    @pl.when(k % 2 == 1)
    def _():
      chunk_impl(k, recb1, posb1, sem1)
    return 0
  lax.fori_loop(0, NCHUNK, chunk, 0)

  for g in range(GPC):
    pltpu.make_async_copy(recb0.at[g], r_hbm.at[posb0.at[g]], sem0).wait()
  for g in range(GPC):
    pltpu.make_async_copy(recb1.at[g], r_hbm.at[posb1.at[g]], sem1).wait()


def _p3_body(r_hbm, rs_hbm, h_hbm, osum_hbm, omax_hbm, odeg_hbm,
             accs, accm, degb, tagb, recb0, recb1, ulb0, ulb1, fb0, fb1,
             nbb0, nbb1, rowsb0, rowsb1, rsl, gsem0, gsem1, rsem0, rsem1):
  w = _wid()
  lo = w * RPW

  _fill_f32(accs, (RPW + 1) * D, 0.0)
  _fill_f32(accm, (RPW + 1) * D, NEG)
  _fill_f32(degb, 320, 0.0)
  _fill_i32(tagb, NP, -1)

  pltpu.sync_copy(rs_hbm, rsl)
  q = pl.multiple_of((w // 16) * 16, 16)
  lane = w % 16
  start = _extract(rsl[0, pl.ds(q, 16)], lane)
  end = _extract(rsl[1, pl.ds(q, 16)], lane)
  abase = pl.multiple_of((start // 8) * 8, 8)
  nblk = (end - abase + RB - 1) // RB

  def issue_rec_impl(b, recb, rsem):
    base = pl.multiple_of(abase + b * RB, 8)
    pltpu.async_copy(r_hbm.at[pl.ds(base, RB)], recb, rsem)

  def issue_rec(b):
    @pl.when(b % 2 == 0)
    def _():
      issue_rec_impl(b, recb0, rsem0)

    @pl.when(b % 2 == 1)
    def _():
      issue_rec_impl(b, recb1, rsem1)

  def stage_impl(b, recb, rsem, ulb, fb, nbb, rowsb, gsem):
    base = pl.multiple_of(abase + b * RB, 8)
    pltpu.make_async_copy(r_hbm.at[pl.ds(base, RB)], recb, rsem).wait()

    def unpack(i, _):
      rec = recb[pl.ds(i * 16, 16)]
      pid = base + i * 16 + _iota16()
      valid = (pid >= start) & (pid < end)
      nb = jnp.where(valid, (rec >> 1) & 16383, 0)
      row = rec >> 15
      ul = jnp.where(valid, row - lo, RPW)
      key = rec >> 1
      cnt1, _unused = plsc.scan_count(key)
      tag = plsc.load_gather(tagb, [nb])
      first = jnp.where(valid & (tag != row) & (cnt1 == 1), 1.0, 0.0)
      plsc.store_scatter(tagb, [nb], row, mask=valid)
      ulb[pl.ds(i * 16, 16)] = ul
      fb[pl.ds(i * 16, 16)] = first
      nbb[pl.ds(i * 16, 16)] = nb
      cntd, lastd = plsc.scan_count(ul, mask=first > 0.0)
      old = plsc.load_gather(degb, [ul], mask=lastd)
      plsc.store_scatter(degb, [ul], old + cntd.astype(jnp.float32),
                         mask=lastd)
      return 0
    lax.fori_loop(0, RB // 16, unpack, 0)
    pltpu.async_copy(h_hbm.at[nbb], rowsb, gsem)

  def stage(b):
    @pl.when(b % 2 == 0)
    def _():
      stage_impl(b, recb0, rsem0, ulb0, fb0, nbb0, rowsb0, gsem0)

    @pl.when(b % 2 == 1)
    def _():
      stage_impl(b, recb1, rsem1, ulb1, fb1, nbb1, rowsb1, gsem1)

  def acc_impl(carry, ulb, fb, nbb, rowsb, gsem):
    pltpu.make_async_copy(h_hbm.at[nbb], rowsb, gsem).wait()

    def rec_step(r, c):
      prev = c[0]
      ul = ulb[pl.ds(r, 16)][0]
      f = fb[pl.ds(r, 16)][0]
      changed = ul != prev

      def do_flush(ops):
        pv = ops[0]
        ab = pv * D
        for j in range(D // 16):
          accs[pl.ds(ab + j * 16, 16)] = ops[1 + j]
          accm[pl.ds(ab + j * 16, 16)] = ops[9 + j]
        return (tuple(jnp.zeros((16,), jnp.float32) for _ in range(8))
                + tuple(jnp.full((16,), NEG, jnp.float32) for _ in range(8)))

      def no_flush(ops):
        return ops[1:]

      regs = lax.cond(changed, do_flush, no_flush, (prev,) + c[1:])
      out = [ul]
      for j in range(D // 16):
        v = rowsb[r, pl.ds(j * 16, 16)]
        out.append(regs[j] + v * f)
      for j in range(D // 16):
        v = rowsb[r, pl.ds(j * 16, 16)]
        out.append(jnp.maximum(regs[8 + j], v))
      return tuple(out)
    return lax.fori_loop(0, RB, rec_step, carry, unroll=2)

  def accumulate(b, carry):
    return lax.cond(
        b % 2 == 0,
        lambda c: acc_impl(c, ulb0, fb0, nbb0, rowsb0, gsem0),
        lambda c: acc_impl(c, ulb1, fb1, nbb1, rowsb1, gsem1),
        carry)

  init = ((jnp.int32(RPW),)
          + tuple(jnp.zeros((16,), jnp.float32) for _ in range(8))
          + tuple(jnp.full((16,), NEG, jnp.float32) for _ in range(8)))

  @pl.when(nblk > 0)
  def _():
    issue_rec(0)

    @pl.when(nblk > 1)
    def _():
      issue_rec(1)
    stage(0)

  def block(b, carry):
    @pl.when(b + 2 < nblk)
    def _():
      issue_rec(b + 2)

    @pl.when(b + 1 < nblk)
    def _():
      stage(b + 1)
    return accumulate(b, carry)
  fin = lax.fori_loop(0, nblk, block, init)

  ab = fin[0] * D
  for j in range(D // 16):
    accs[pl.ds(ab + j * 16, 16)] = fin[1 + j]
    accm[pl.ds(ab + j * 16, 16)] = fin[9 + j]

  pltpu.sync_copy(accs.at[pl.ds(0, RPW * D)],
                  osum_hbm.at[pl.ds(lo * D, RPW * D)])
  pltpu.sync_copy(accm.at[pl.ds(0, RPW * D)],
                  omax_hbm.at[pl.ds(lo * D, RPW * D)])
  pltpu.sync_copy(degb, odeg_hbm.at[w])


def _p4_body(h_ref, s_ref, m_ref, deg_ref, wa_ref, ba_ref,
             w1_ref, b1_ref, w2_ref, b2_ref, w3_ref, b3_ref, o_ref):
  deg = deg_ref[...]
  has = deg > 0.0
  s = jnp.where(has, s_ref[...], 0.0)
  m = jnp.where(has, m_ref[...], 0.0)
  wa = wa_ref[...]
  dn = (((1,), (0,)), ((), ()))
  f32 = jnp.float32
  hi = lax.Precision.DEFAULT
  sc0 = (lax.dot_general(s, wa[0, :D], dn, preferred_element_type=f32,
                         precision=hi)
         + lax.dot_general(m, wa[0, D:], dn, preferred_element_type=f32,
                           precision=hi) + ba_ref[0])
  sc1 = (lax.dot_general(s, wa[1, :D], dn, preferred_element_type=f32,
                         precision=hi)
         + lax.dot_general(m, wa[1, D:], dn, preferred_element_type=f32,
                           precision=hi) + ba_ref[1])
  w0 = 1.0 / (1.0 + jnp.exp(sc1 - sc0))
  w1w = 1.0 - w0
  x = h_ref[...] + w0[:, None] * s + w1w[:, None] * m
  dnm = (((1,), (1,)), ((), ()))
  x = jnp.maximum(
      lax.dot_general(x, w1_ref[...], dnm, preferred_element_type=f32,
                      precision=hi) + b1_ref[...][None], 0.0)
  x = jnp.maximum(
      lax.dot_general(x, w2_ref[...], dnm, preferred_element_type=f32,
                      precision=hi) + b2_ref[...][None], 0.0)
  o_ref[...] = (
      lax.dot_general(x, w3_ref[...], dnm, preferred_element_type=f32,
                      precision=hi) + b3_ref[...][None])


def kernel(h, edge_index, Wa, ba, W1, b1, W2, b2, W3, b3):
  src = edge_index[0]
  dst = edge_index[1]

  p1 = pl.kernel(
      _p1_body,
      out_type=jax.ShapeDtypeStruct((NW, NP), jnp.int32),
      mesh=_mesh,
      compiler_params=_cp,
      scratch_types=[
          pltpu.VMEM((CH,), jnp.int32),
          pltpu.VMEM((CH,), jnp.int32),
          pltpu.VMEM((NP,), jnp.int32),
      ],
  )
  hist = p1(src, dst)

  p2 = pl.kernel(
      _p2_body,
      out_type=(
          jax.ShapeDtypeStruct((RPAD,), jnp.int32),
          jax.ShapeDtypeStruct((2, NW), jnp.int32),
      ),
      mesh=_mesh,
      compiler_params=_cp,
      scratch_types=[
          pltpu.VMEM((CH,), jnp.int32),
          pltpu.VMEM((CH,), jnp.int32),
          pltpu.VMEM((GPC, DB), jnp.int32),
          pltpu.VMEM((GPC, DB), jnp.int32),
          pltpu.VMEM((GPC, DB), jnp.int32),
          pltpu.VMEM((GPC, DB), jnp.int32),
          pltpu.VMEM((NP,), jnp.int32),
          pltpu.VMEM((NP,), jnp.int32),
          pltpu.VMEM((NP,), jnp.int32),
          pltpu.VMEM((NP,), jnp.int32),
          pltpu.VMEM((NP,), jnp.int32),
          pltpu.VMEM((2, NW), jnp.int32),
          pltpu.SemaphoreType.DMA,
          pltpu.SemaphoreType.DMA,
          pltpu.SemaphoreType.DMA,
          pltpu.SemaphoreType.DMA,
      ],
  )
  recs, rbounds = p2(src, dst, hist)

  p3 = pl.kernel(
      _p3_body,
      out_type=(
          jax.ShapeDtypeStruct((NP * D,), jnp.float32),
          jax.ShapeDtypeStruct((NP * D,), jnp.float32),
          jax.ShapeDtypeStruct((NW, 320), jnp.float32),
      ),
      mesh=_mesh,
      compiler_params=_cp,
      scratch_types=[
          pltpu.VMEM(((RPW + 1) * D,), jnp.float32),
          pltpu.VMEM(((RPW + 1) * D,), jnp.float32),
          pltpu.VMEM((320,), jnp.float32),
          pltpu.VMEM((NP,), jnp.int32),
          pltpu.VMEM((RB,), jnp.int32),
          pltpu.VMEM((RB,), jnp.int32),
          pltpu.VMEM((RB + 16,), jnp.int32),
          pltpu.VMEM((RB + 16,), jnp.int32),
          pltpu.VMEM((RB + 16,), jnp.float32),
          pltpu.VMEM((RB + 16,), jnp.float32),
          pltpu.VMEM((RB,), jnp.int32),
          pltpu.VMEM((RB,), jnp.int32),
          pltpu.VMEM((RB, D), jnp.float32),
          pltpu.VMEM((RB, D), jnp.float32),
          pltpu.VMEM((2, NW), jnp.int32),
          pltpu.SemaphoreType.DMA,
          pltpu.SemaphoreType.DMA,
          pltpu.SemaphoreType.DMA,
          pltpu.SemaphoreType.DMA,
      ],
  )
  osum, omax, odeg = p3(recs, rbounds, h)

  asum = osum.reshape(NP, D)[:N]
  amax = omax.reshape(NP, D)[:N]
  deg = odeg[:, :RPW].reshape(NP)[:N].reshape(N, 1)

  grid = 25
  blk = N // grid
  out = pl.pallas_call(
      _p4_body,
      out_shape=jax.ShapeDtypeStruct((N, D), jnp.float32),
      grid=(grid,),
      in_specs=[
          pl.BlockSpec((blk, D), lambda i: (i, 0)),
          pl.BlockSpec((blk, D), lambda i: (i, 0)),
          pl.BlockSpec((blk, D), lambda i: (i, 0)),
          pl.BlockSpec((blk, 1), lambda i: (i, 0)),
          pl.BlockSpec((2, 2 * D), lambda i: (0, 0)),
          pl.BlockSpec((2,), lambda i: (0,)),
          pl.BlockSpec((D, D), lambda i: (0, 0)),
          pl.BlockSpec((D,), lambda i: (0,)),
          pl.BlockSpec((D, D), lambda i: (0, 0)),
          pl.BlockSpec((D,), lambda i: (0,)),
          pl.BlockSpec((D, D), lambda i: (0, 0)),
          pl.BlockSpec((D,), lambda i: (0,)),
      ],
      out_specs=pl.BlockSpec((blk, D), lambda i: (i, 0)),
  )(h, asum, amax, deg, Wa, ba, W1, b1, W2, b2, W3, b3)
  return out


# speedup vs baseline: 1.4601x; 1.4601x over previous
import jax
import jax.numpy as jnp
from jax import lax
from jax.experimental import pallas as pl
from jax.experimental.pallas import tpu as pltpu
from jax.experimental.pallas import tpu_sc as plsc

N = 10000
D = 128
E = 320000
E2 = 2 * E
NC = 2
NS = 16
NW = NC * NS
EPW = E2 // NW
RPW = 313
NP = NW * RPW
CH = 2000
NCHUNK = EPW // CH
DB = 80
GPC = CH // DB
VPG = DB // 16
RB = 128
RPAD = E2 + 2 * RB
NEG = -3.0e38

_mesh = plsc.VectorSubcoreMesh(core_axis_name="c", subcore_axis_name="s")
_cp = pltpu.CompilerParams(needs_layout_passes=False)


def _wid():
  return lax.axis_index("s") * NC + lax.axis_index("c")


def _iota16():
  return lax.iota(jnp.int32, 16)


def _extract(vec, pos):
  return jnp.sum(jnp.where(_iota16() == pos, vec, 0))


def _zero_i32(ref, nwords):
  def body(i, _):
    ref[pl.ds(i * 16, 16)] = jnp.zeros((16,), jnp.int32)
    return 0
  lax.fori_loop(0, nwords // 16, body, 0)


def _fill_f32(ref, nwords, val):
  def body(i, _):
    ref[pl.ds(i * 16, 16)] = jnp.full((16,), val, jnp.float32)
    return 0
  lax.fori_loop(0, nwords // 16, body, 0)


def _fill_i32(ref, nwords, val):
  def body(i, _):
    ref[pl.ds(i * 16, 16)] = jnp.full((16,), val, jnp.int32)
    return 0
  lax.fori_loop(0, nwords // 16, body, 0)


def _p1_body(src_hbm, dst_hbm, hist_hbm, sbuf, dbuf, hbuf):
  w = _wid()
  swap = w >= 16
  jbase = (w % 16) * EPW

  _zero_i32(hbuf, NP)

  def chunk(k, _):
    off = jbase + k * CH
    pltpu.sync_copy(src_hbm.at[pl.ds(off, CH)], sbuf)
    pltpu.sync_copy(dst_hbm.at[pl.ds(off, CH)], dbuf)

    def vreg(i, _):
      s = sbuf[pl.ds(i * 16, 16)]
      d = dbuf[pl.ds(i * 16, 16)]
      row = jnp.where(swap, d, s)
      cnt, last = plsc.scan_count(row)
      old = plsc.load_gather(hbuf, [row], mask=last)
      plsc.store_scatter(hbuf, [row], old + cnt, mask=last)
      return 0
    lax.fori_loop(0, CH // 16, vreg, 0)
    return 0
  lax.fori_loop(0, NCHUNK, chunk, 0)

  pltpu.sync_copy(hbuf, hist_hbm.at[w])


def _p2_body(src_hbm, dst_hbm, hist_hbm, r_hbm, rs_hbm,
             sbuf, dbuf, recb0, recb1, posb0, posb1, tbuf, cbuf, pbuf,
             tmpb0, tmpb1, rs2, rsp, sem0, sem1):
  c = lax.axis_index("c")
  sax = lax.axis_index("s")
  jbase = sax * EPW
  HALF = 313 * 16

  _zero_i32(tbuf, NP)
  _zero_i32(cbuf, NP)

  for wp in range(16):
    pltpu.sync_copy(hist_hbm.at[wp], tmpb0)
    pltpu.sync_copy(hist_hbm.at[wp + 16], tmpb1)

    def addrow(i, _):
      t = tmpb0[pl.ds(i * 16, 16)] + tmpb1[pl.ds(i * 16, 16)]
      tbuf[pl.ds(i * 16, 16)] = tbuf[pl.ds(i * 16, 16)] + t
      cbuf[pl.ds(i * 16, 16)] = cbuf[pl.ds(i * 16, 16)] + jnp.where(
          wp < sax, t, 0)
      return 0
    lax.fori_loop(0, NP // 16, addrow, 0, unroll=4)

  def prefix(i, carry):
    t = tbuf[pl.ds(i * 16, 16)]
    cs = plsc.cumsum(t)
    excl = cs - t + carry
    pbuf[pl.ds(i * 16, 16)] = excl
    cbuf[pl.ds(i * 16, 16)] = cbuf[pl.ds(i * 16, 16)] + excl
    return carry + jnp.max(cs)
  lax.fori_loop(0, NP // 16, prefix, 0)

  l0 = pbuf[pl.ds(HALF, 16)][0]
  base1 = ((l0 + 2047) // 2048) * 2048
  gap = base1 - l0

  @pl.when((c == 0) & (sax == 0))
  def _():
    io = _iota16()
    for half in range(2):
      starts = plsc.load_gather(pbuf, [(io + half * 16) * RPW])
      nxt = jnp.minimum((io + half * 16 + 1) * RPW, NP - 1)
      ends = plsc.load_gather(pbuf, [nxt])
      if half == 1:
        starts = starts + gap
        ends = jnp.where(io == 15, E2 + gap, ends + gap)
      rs2[0, pl.ds(half * 16, 16)] = starts
      rs2[1, pl.ds(half * 16, 16)] = ends
    pltpu.sync_copy(rs2, rs_hbm)

  def gapadj(i, _):
    sl = pl.ds((313 + i) * 16, 16)
    cbuf[sl] = cbuf[sl] + gap
    return 0
  lax.fori_loop(0, NP // 16 - 313, gapadj, 0)

  regionbase = jnp.where(c == 1, base1, 0)

  def step_impl(k, r, recb, posb, sem, first_steps):
    @pl.when(r == 0)
    def _():
      off = jbase + k * CH
      pltpu.sync_copy(src_hbm.at[pl.ds(off, CH)], sbuf)
      pltpu.sync_copy(dst_hbm.at[pl.ds(off, CH)], dbuf)

    @pl.when(jnp.logical_not(first_steps))
    def _():
      for g in range(GPC):
        pltpu.make_async_copy(recb.at[g], rsp.at[posb.at[g]], sem).wait()

    for g in range(GPC):
      def vreg(v, _):
        i = g * VPG + v
        s = sbuf[pl.ds(i * 16, 16)]
        d = dbuf[pl.ds(i * 16, 16)]
        row = jnp.where(r == 1, d, s)
        nb = jnp.where(r == 1, s, d)
        match = jnp.where(c == 1, row >= HALF, row < HALF)
        recb[g, pl.ds(v * 16, 16)] = row * 32768 + nb * 2
        cnt, last = plsc.scan_count(row, mask=match)
        base = plsc.load_gather(cbuf, [row])
        pos = base + cnt - 1 - regionbase
        posb[g, pl.ds(v * 16, 16)] = jnp.where(match, pos, E2)
        plsc.store_scatter(cbuf, [row], base + cnt, mask=last)
        return 0
      lax.fori_loop(0, VPG, vreg, 0)

    for g in range(GPC):
      pltpu.async_copy(recb.at[g], rsp.at[posb.at[g]], sem)

  def step(t, _):
    k = t // 2
    r = t % 2

    @pl.when(t % 2 == 0)
    def _():
      step_impl(k, r, recb0, posb0, sem0, t < 2)

    @pl.when(t % 2 == 1)
    def _():
      step_impl(k, r, recb1, posb1, sem1, t < 2)
    return 0
  lax.fori_loop(0, 2 * NCHUNK, step, 0)

  for g in range(GPC):
    pltpu.make_async_copy(recb0.at[g], rsp.at[posb0.at[g]], sem0).wait()
  for g in range(GPC):
    pltpu.make_async_copy(recb1.at[g], rsp.at[posb1.at[g]], sem1).wait()

  plsc.subcore_barrier()

  lenc = jnp.where(c == 1, E2 - l0, l0)
  nch = (lenc + 2047) // 2048

  def copyout(t, _):
    cid = sax + t * 16

    @pl.when(cid < nch)
    def _():
      off = pl.multiple_of(cid * 2048, 2048)
      dst = pl.multiple_of(regionbase + cid * 2048, 2048)
      pltpu.sync_copy(rsp.at[pl.ds(off, 2048)], r_hbm.at[pl.ds(dst, 2048)])
    return 0
  lax.fori_loop(0, (E2 + 2048) // 2048 // 16 + 1, copyout, 0)


def _p3_body(r_hbm, rs_hbm, h_hbm, osum_hbm, omax_hbm, odeg_hbm,
             accs, accm, degb, tagb, recb0, recb1, ulb0, ulb1, fb0, fb1,
             nbb0, nbb1, rowsb0, rowsb1, rsl, gsem0, gsem1, rsem0, rsem1):
  w = _wid()
  lo = w * RPW

  _fill_f32(accs, (RPW + 1) * D, 0.0)
  _fill_f32(accm, (RPW + 1) * D, NEG)
  _fill_f32(degb, 320, 0.0)
  _fill_i32(tagb, NP, -1)

  pltpu.sync_copy(rs_hbm, rsl)
  q = pl.multiple_of((w // 16) * 16, 16)
  lane = w % 16
  start = _extract(rsl[0, pl.ds(q, 16)], lane)
  end = _extract(rsl[1, pl.ds(q, 16)], lane)
  abase = pl.multiple_of((start // 8) * 8, 8)
  nblk = (end - abase + RB - 1) // RB

  def issue_rec_impl(b, recb, rsem):
    base = pl.multiple_of(abase + b * RB, 8)
    pltpu.async_copy(r_hbm.at[pl.ds(base, RB)], recb, rsem)

  def issue_rec(b):
    @pl.when(b % 2 == 0)
    def _():
      issue_rec_impl(b, recb0, rsem0)

    @pl.when(b % 2 == 1)
    def _():
      issue_rec_impl(b, recb1, rsem1)

  def stage_impl(b, recb, rsem, ulb, fb, nbb, rowsb, gsem):
    base = pl.multiple_of(abase + b * RB, 8)
    pltpu.make_async_copy(r_hbm.at[pl.ds(base, RB)], recb, rsem).wait()

    def unpack(i, _):
      rec = recb[pl.ds(i * 16, 16)]
      pid = base + i * 16 + _iota16()
      valid = (pid >= start) & (pid < end)
      nb = jnp.where(valid, (rec >> 1) & 16383, 0)
      row = rec >> 15
      ul = jnp.where(valid, row - lo, RPW)
      key = rec >> 1
      cnt1, _unused = plsc.scan_count(key)
      tag = plsc.load_gather(tagb, [nb])
      first = jnp.where(valid & (tag != row) & (cnt1 == 1), 1.0, 0.0)
      plsc.store_scatter(tagb, [nb], row, mask=valid)
      ulb[pl.ds(i * 16, 16)] = ul
      fb[pl.ds(i * 16, 16)] = first
      nbb[pl.ds(i * 16, 16)] = nb
      cntd, lastd = plsc.scan_count(ul, mask=first > 0.0)
      old = plsc.load_gather(degb, [ul], mask=lastd)
      plsc.store_scatter(degb, [ul], old + cntd.astype(jnp.float32),
                         mask=lastd)
      return 0
    lax.fori_loop(0, RB // 16, unpack, 0)
    pltpu.async_copy(h_hbm.at[nbb], rowsb, gsem)

  def stage(b):
    @pl.when(b % 2 == 0)
    def _():
      stage_impl(b, recb0, rsem0, ulb0, fb0, nbb0, rowsb0, gsem0)

    @pl.when(b % 2 == 1)
    def _():
      stage_impl(b, recb1, rsem1, ulb1, fb1, nbb1, rowsb1, gsem1)

  def acc_impl(carry, ulb, fb, nbb, rowsb, gsem):
    pltpu.make_async_copy(h_hbm.at[nbb], rowsb, gsem).wait()

    def rec_step(r, c):
      prev = c[0]
      ul = ulb[pl.ds(r, 16)][0]
      f = fb[pl.ds(r, 16)][0]
      changed = ul != prev

      def do_flush(ops):
        pv = ops[0]
        ab = pv * D
        for j in range(D // 16):
          accs[pl.ds(ab + j * 16, 16)] = ops[1 + j]
          accm[pl.ds(ab + j * 16, 16)] = ops[9 + j]
        return (tuple(jnp.zeros((16,), jnp.float32) for _ in range(8))
                + tuple(jnp.full((16,), NEG, jnp.float32) for _ in range(8)))

      def no_flush(ops):
        return ops[1:]

      regs = lax.cond(changed, do_flush, no_flush, (prev,) + c[1:])
      out = [ul]
      for j in range(D // 16):
        v = rowsb[r, pl.ds(j * 16, 16)]
        out.append(regs[j] + v * f)
      for j in range(D // 16):
        v = rowsb[r, pl.ds(j * 16, 16)]
        out.append(jnp.maximum(regs[8 + j], v))
      return tuple(out)
    return lax.fori_loop(0, RB, rec_step, carry, unroll=2)

  def accumulate(b, carry):
    return lax.cond(
        b % 2 == 0,
        lambda c: acc_impl(c, ulb0, fb0, nbb0, rowsb0, gsem0),
        lambda c: acc_impl(c, ulb1, fb1, nbb1, rowsb1, gsem1),
        carry)

  init = ((jnp.int32(RPW),)
          + tuple(jnp.zeros((16,), jnp.float32) for _ in range(8))
          + tuple(jnp.full((16,), NEG, jnp.float32) for _ in range(8)))

  @pl.when(nblk > 0)
  def _():
    issue_rec(0)

    @pl.when(nblk > 1)
    def _():
      issue_rec(1)
    stage(0)

  def block(b, carry):
    @pl.when(b + 2 < nblk)
    def _():
      issue_rec(b + 2)

    @pl.when(b + 1 < nblk)
    def _():
      stage(b + 1)
    return accumulate(b, carry)
  fin = lax.fori_loop(0, nblk, block, init)

  ab = fin[0] * D
  for j in range(D // 16):
    accs[pl.ds(ab + j * 16, 16)] = fin[1 + j]
    accm[pl.ds(ab + j * 16, 16)] = fin[9 + j]

  pltpu.sync_copy(accs.at[pl.ds(0, RPW * D)],
                  osum_hbm.at[pl.ds(lo * D, RPW * D)])
  pltpu.sync_copy(accm.at[pl.ds(0, RPW * D)],
                  omax_hbm.at[pl.ds(lo * D, RPW * D)])
  pltpu.sync_copy(degb, odeg_hbm.at[w])


def _p4_body(h_ref, s_ref, m_ref, deg_ref, wa_ref, ba_ref,
             w1_ref, b1_ref, w2_ref, b2_ref, w3_ref, b3_ref, o_ref):
  deg = deg_ref[...]
  has = deg > 0.0
  s = jnp.where(has, s_ref[...], 0.0)
  m = jnp.where(has, m_ref[...], 0.0)
  wa = wa_ref[...]
  dn = (((1,), (0,)), ((), ()))
  f32 = jnp.float32
  hi = lax.Precision.DEFAULT
  sc0 = (lax.dot_general(s, wa[0, :D], dn, preferred_element_type=f32,
                         precision=hi)
         + lax.dot_general(m, wa[0, D:], dn, preferred_element_type=f32,
                           precision=hi) + ba_ref[0])
  sc1 = (lax.dot_general(s, wa[1, :D], dn, preferred_element_type=f32,
                         precision=hi)
         + lax.dot_general(m, wa[1, D:], dn, preferred_element_type=f32,
                           precision=hi) + ba_ref[1])
  w0 = 1.0 / (1.0 + jnp.exp(sc1 - sc0))
  w1w = 1.0 - w0
  x = h_ref[...] + w0[:, None] * s + w1w[:, None] * m
  dnm = (((1,), (1,)), ((), ()))
  x = jnp.maximum(
      lax.dot_general(x, w1_ref[...], dnm, preferred_element_type=f32,
                      precision=hi) + b1_ref[...][None], 0.0)
  x = jnp.maximum(
      lax.dot_general(x, w2_ref[...], dnm, preferred_element_type=f32,
                      precision=hi) + b2_ref[...][None], 0.0)
  o_ref[...] = (
      lax.dot_general(x, w3_ref[...], dnm, preferred_element_type=f32,
                      precision=hi) + b3_ref[...][None])


def kernel(h, edge_index, Wa, ba, W1, b1, W2, b2, W3, b3):
  src = edge_index[0]
  dst = edge_index[1]

  p1 = pl.kernel(
      _p1_body,
      out_type=jax.ShapeDtypeStruct((NW, NP), jnp.int32),
      mesh=_mesh,
      compiler_params=_cp,
      scratch_types=[
          pltpu.VMEM((CH,), jnp.int32),
          pltpu.VMEM((CH,), jnp.int32),
          pltpu.VMEM((NP,), jnp.int32),
      ],
  )
  hist = p1(src, dst)

  p2 = pl.kernel(
      _p2_body,
      out_type=(
          jax.ShapeDtypeStruct((RPAD,), jnp.int32),
          jax.ShapeDtypeStruct((2, NW), jnp.int32),
      ),
      mesh=_mesh,
      compiler_params=_cp,
      scratch_types=[
          pltpu.VMEM((CH,), jnp.int32),
          pltpu.VMEM((CH,), jnp.int32),
          pltpu.VMEM((GPC, DB), jnp.int32),
          pltpu.VMEM((GPC, DB), jnp.int32),
          pltpu.VMEM((GPC, DB), jnp.int32),
          pltpu.VMEM((GPC, DB), jnp.int32),
          pltpu.VMEM((NP,), jnp.int32),
          pltpu.VMEM((NP,), jnp.int32),
          pltpu.VMEM((NP,), jnp.int32),
          pltpu.VMEM((NP,), jnp.int32),
          pltpu.VMEM((NP,), jnp.int32),
          pltpu.VMEM((2, NW), jnp.int32),
          pltpu.VMEM_SHARED((E2 + 2048,), jnp.int32),
          pltpu.SemaphoreType.DMA,
          pltpu.SemaphoreType.DMA,
      ],
  )
  recs, rbounds = p2(src, dst, hist)

  p3 = pl.kernel(
      _p3_body,
      out_type=(
          jax.ShapeDtypeStruct((NP * D,), jnp.float32),
          jax.ShapeDtypeStruct((NP * D,), jnp.float32),
          jax.ShapeDtypeStruct((NW, 320), jnp.float32),
      ),
      mesh=_mesh,
      compiler_params=_cp,
      scratch_types=[
          pltpu.VMEM(((RPW + 1) * D,), jnp.float32),
          pltpu.VMEM(((RPW + 1) * D,), jnp.float32),
          pltpu.VMEM((320,), jnp.float32),
          pltpu.VMEM((NP,), jnp.int32),
          pltpu.VMEM((RB,), jnp.int32),
          pltpu.VMEM((RB,), jnp.int32),
          pltpu.VMEM((RB + 16,), jnp.int32),
          pltpu.VMEM((RB + 16,), jnp.int32),
          pltpu.VMEM((RB + 16,), jnp.float32),
          pltpu.VMEM((RB + 16,), jnp.float32),
          pltpu.VMEM((RB,), jnp.int32),
          pltpu.VMEM((RB,), jnp.int32),
          pltpu.VMEM((RB, D), jnp.float32),
          pltpu.VMEM((RB, D), jnp.float32),
          pltpu.VMEM((2, NW), jnp.int32),
          pltpu.SemaphoreType.DMA,
          pltpu.SemaphoreType.DMA,
          pltpu.SemaphoreType.DMA,
          pltpu.SemaphoreType.DMA,
      ],
  )
  osum, omax, odeg = p3(recs, rbounds, h)

  asum = osum.reshape(NP, D)[:N]
  amax = omax.reshape(NP, D)[:N]
  deg = odeg[:, :RPW].reshape(NP)[:N].reshape(N, 1)

  grid = 25
  blk = N // grid
  out = pl.pallas_call(
      _p4_body,
      out_shape=jax.ShapeDtypeStruct((N, D), jnp.float32),
      grid=(grid,),
      in_specs=[
          pl.BlockSpec((blk, D), lambda i: (i, 0)),
          pl.BlockSpec((blk, D), lambda i: (i, 0)),
          pl.BlockSpec((blk, D), lambda i: (i, 0)),
          pl.BlockSpec((blk, 1), lambda i: (i, 0)),
          pl.BlockSpec((2, 2 * D), lambda i: (0, 0)),
          pl.BlockSpec((2,), lambda i: (0,)),
          pl.BlockSpec((D, D), lambda i: (0, 0)),
          pl.BlockSpec((D,), lambda i: (0,)),
          pl.BlockSpec((D, D), lambda i: (0, 0)),
          pl.BlockSpec((D,), lambda i: (0,)),
          pl.BlockSpec((D, D), lambda i: (0, 0)),
          pl.BlockSpec((D,), lambda i: (0,)),
      ],
      out_specs=pl.BlockSpec((blk, D), lambda i: (i, 0)),
  )(h, asum, amax, deg, Wa, ba, W1, b1, W2, b2, W3, b3)
  return out
